# baseline (speedup 1.0000x reference)
import math
import functools

import jax
import jax.numpy as jnp
from jax import lax
from jax.experimental import pallas as pl
from jax.experimental.pallas import tpu as pltpu
from jax.experimental.pallas import tpu_sc as plsc

NC = 2
NS = 16
NW = NC * NS
LANES = 16

CHUNK = 128
GROUPS = CHUNK // LANES


def _tc_repack(Wt, V, D):
  VB = 1024
  grid = pl.cdiv(V, VB)

  def body(x_ref, out_ref):
    x = x_ref[...]
    out_ref[:, 0:D] = x[:, 0:VB // 2].T
    out_ref[:, D:2 * D] = x[:, VB // 2:VB].T

  return pl.pallas_call(
      body,
      grid=(grid,),
      in_specs=[pl.BlockSpec((D, VB), lambda i: (0, i))],
      out_specs=pl.BlockSpec((VB // 2, 2 * D), lambda i: (i, 0)),
      out_shape=jax.ShapeDtypeStruct((grid * VB // 2, 2 * D), jnp.float32),
  )(Wt)


def _make_sc_kernel(B, L, V2, K, D):
  N = B * L
  NBLK = N // CHUNK
  BB = B // CHUNK
  NB = NBLK // NW
  DT = D // 8
  assert NBLK % NW == 0 and NB % 2 == 0

  mesh = plsc.VectorSubcoreMesh(
      core_axis_name="c", subcore_axis_name="s", num_cores=NC,
      num_subcores=NS)

  @functools.partial(
      pl.kernel,
      out_type=jax.ShapeDtypeStruct((N * D,), jnp.float32),
      mesh=mesh,
      scratch_types=[
          pltpu.VMEM((2, CHUNK), jnp.int32),
          pltpu.VMEM((2, CHUNK), jnp.int32),
          pltpu.VMEM((2, CHUNK), jnp.int32),
          pltpu.VMEM((2, CHUNK, D), jnp.float32),
          pltpu.VMEM((2, CHUNK, LANES), jnp.float32),
          pltpu.VMEM((2, DT, 8 * CHUNK), jnp.float32),
          pltpu.VMEM((LANES,), jnp.float32),
          pltpu.SemaphoreType.DMA,
          pltpu.SemaphoreType.DMA,
          pltpu.SemaphoreType.DMA,
          pltpu.SemaphoreType.DMA,
      ],
      compiler_params=pltpu.CompilerParams(
          needs_layout_passes=False, use_tc_tiling_on_sc=False),
  )
  def sc_kernel(ids_hbm, w_hbm, mix_hbm, scp_hbm, out_hbm,
                idx_v, idw_v, idq_v, rows_v, mixr_v, outt_v, sc_v,
                sem0, sem1, osem0, osem1):
    sems = (sem0, sem1)
    osems = (osem0, osem1)
    wid = lax.axis_index("s") * NC + lax.axis_index("c")
    pltpu.sync_copy(scp_hbm, sc_v)
    scv = sc_v[...]
    s_val = [scv[k] for k in range(K)]

    def gather_copies(slot):
      return [
          pltpu.make_async_copy(
              w_hbm.at[idw_v.at[slot]], rows_v.at[slot], sems[slot]),
          pltpu.make_async_copy(
              mix_hbm.at[idq_v.at[slot]], mixr_v.at[slot], sems[slot]),
      ]

    def out_copies(g, slot):
      blk = wid * NB + g
      l = blk // BB
      bb = blk % BB
      obase = l * (D * B) + bb * (8 * CHUNK)
      return [
          pltpu.make_async_copy(
              outt_v.at[slot, dt],
              out_hbm.at[pl.ds(obase + dt * (8 * B), 8 * CHUNK)],
              osems[slot])
          for dt in range(DT)
      ]

    def fire(g, slot):
      blk = wid * NB + g
      l = blk // BB
      bb = blk % BB
      o = (l // 8) * (8 * B) + bb * (8 * CHUNK) + (l % 8) * CHUNK
      pltpu.sync_copy(ids_hbm.at[pl.ds(o, CHUNK)], idx_v.at[slot])

      def shift_body(i, carry):
        sl = pl.ds(i * LANES, LANES)
        raw = idx_v[slot, sl]
        idw_v[slot, sl] = ((raw & -1024)
                           + lax.shift_left(raw & 511, 1)
                           + (lax.shift_right_logical(raw, 9) & 1))
        idq_v[slot, sl] = lax.shift_right_logical(raw, 1)
        return carry

      lax.fori_loop(0, GROUPS, shift_body, 0)
      for c in gather_copies(slot):
        c.start()

    def drain(slot):
      for c in gather_copies(slot):
        c.wait()

    def compute(g, slot):
      rows = rows_v.at[slot]
      mixr = mixr_v.at[slot]

      def group_body(i, carry):
        t0 = i * LANES
        tok = t0 + lax.iota(jnp.int32, LANES)
        idvec = idx_v[slot, pl.ds(i * LANES, LANES)]
        colb = (idvec & 1) * 8
        logits = [plsc.load_gather(mixr, [tok, colb + k]) for k in range(K)]
        m = logits[0]
        for k in range(1, K):
          m = jnp.maximum(m, logits[k])
        e = [jnp.exp(logits[k] - m) for k in range(K)]
        tot = e[0]
        for k in range(1, K):
          tot = tot + e[k]
        inv = 1.0 / tot
        wk = [e[k] * inv * s_val[k] for k in range(K)]
        for k in range(K):
          for j in range(LANES):
            dcol = k * LANES + j
            vals = plsc.load_gather(
                rows, [tok, jnp.full((LANES,), dcol, jnp.int32)])
            dt, dr = dcol // 8, dcol % 8
            outt_v[slot, dt, pl.ds(dr * CHUNK + t0, LANES)] = vals * wk[k]
        return carry

      lax.fori_loop(0, GROUPS, group_body, 0)
      for c in out_copies(g, slot):
        c.start()

    def drain_out(g, slot):
      for c in out_copies(g, slot):
        c.wait()

    fire(0, 0)
    fire(1, 1)

    def pair_body(p, carry):
      g0 = 2 * p
      drain(0)
      compute(g0, 0)
      fire(g0 + 2, 0)
      drain(1)
      compute(g0 + 1, 1)
      fire(g0 + 3, 1)
      drain_out(g0, 0)
      drain_out(g0 + 1, 1)
      return carry

    lax.fori_loop(0, NB // 2 - 1, pair_body, 0)
    drain(0)
    compute(NB - 2, 0)
    drain(1)
    compute(NB - 1, 1)
    drain_out(NB - 2, 0)
    drain_out(NB - 1, 1)

  return sc_kernel


def kernel(ids, W, mix, sc):
  B, L = ids.shape
  V, K, d = W.shape
  D = K * d
  assert V % 4 == 0 and K == 4 and D == 64 and B % 128 == 0 and L % 8 == 0
  ids_nat = (ids.astype(jnp.int32).T.reshape(L // 8, 8, B // 128, 128)
             .transpose(0, 2, 1, 3).reshape(B * L))
  Wt = jnp.transpose(W, (1, 2, 0)).reshape(D, V)
  Wrp = _tc_repack(Wt, V, D)
  W2 = Wrp.reshape(Wrp.shape[0] * 2, D)
  mix_pad = jnp.pad(mix, ((0, 0), (0, 8 - K)))
  mix128 = lax.optimization_barrier(mix_pad.reshape(V * 8 // 128, 128))
  mix16 = mix128.reshape(V // 2, 16)
  scp = jnp.zeros((LANES,), jnp.float32).at[:K].set(
      sc.astype(jnp.float32) * math.sqrt(D))
  flat = _make_sc_kernel(B, L, W2.shape[0], K, D)(ids_nat, W2, mix16, scp)
  X = flat.reshape(L, D // 8, B // 128, 8, 128)
  return X.transpose(2, 4, 0, 1, 3).reshape(B, L, D)


# speedup vs baseline: 1.2602x; 1.2602x over previous
import math
import functools

import jax
import jax.numpy as jnp
from jax import lax
from jax.experimental import pallas as pl
from jax.experimental.pallas import tpu as pltpu
from jax.experimental.pallas import tpu_sc as plsc

NC = 2
NS = 16
NW = NC * NS
LANES = 16

CHUNK = 640
SUB = CHUNK // 128
GROUPS = CHUNK // LANES


_GATHER_DNUMS = lax.GatherDimensionNumbers(
    offset_dims=(), collapsed_slice_dims=(0,), start_index_map=(0,))


def _tc_repack(Wt, V, D):
  VB = 1024
  grid = pl.cdiv(V, VB)

  def body(x_ref, out_ref):
    x = x_ref[...]
    out_ref[:, 0:D] = x[:, 0:VB // 2].T
    out_ref[:, D:2 * D] = x[:, VB // 2:VB].T

  return pl.pallas_call(
      body,
      grid=(grid,),
      in_specs=[pl.BlockSpec((D, VB), lambda i: (0, i))],
      out_specs=pl.BlockSpec((VB // 2, 2 * D), lambda i: (i, 0)),
      out_shape=jax.ShapeDtypeStruct((grid * VB // 2, 2 * D), jnp.float32),
  )(Wt)


def _lane_bcast(v, lane):
  idx = jnp.full((LANES, 1), lane, dtype=jnp.int32)
  return lax.gather(v, idx, _GATHER_DNUMS, (1,),
                    mode=lax.GatherScatterMode.PROMISE_IN_BOUNDS)


def _make_sc_kernel(N, V, K, D):
  TW = N // NW
  NCHUNK = TW // CHUNK
  assert TW % CHUNK == 0 and CHUNK % 128 == 0 and D % LANES == 0

  mesh = plsc.VectorSubcoreMesh(
      core_axis_name="c", subcore_axis_name="s", num_cores=NC,
      num_subcores=NS)

  @functools.partial(
      pl.kernel,
      out_type=jax.ShapeDtypeStruct((N, D), jnp.float32),
      mesh=mesh,
      scratch_types=[
          pltpu.VMEM((2, CHUNK), jnp.int32),
          pltpu.VMEM((2, CHUNK), jnp.int32),
          pltpu.VMEM((2, CHUNK), jnp.int32),
          pltpu.VMEM((2, CHUNK, D), jnp.float32),
          pltpu.VMEM((2, CHUNK, LANES), jnp.float32),
          pltpu.VMEM((LANES,), jnp.float32),
          pltpu.SemaphoreType.DMA,
          pltpu.SemaphoreType.DMA,
      ],
      compiler_params=pltpu.CompilerParams(
          needs_layout_passes=False, use_tc_tiling_on_sc=False),
  )
  def sc_kernel(ids_hbm, w_hbm, mix_hbm, scp_hbm, out_hbm,
                idx_v, idw_v, idq_v, rows_v, mixr_v, sc_v, sem0, sem1):
    sems = (sem0, sem1)
    wid = lax.axis_index("s") * NC + lax.axis_index("c")
    pltpu.sync_copy(scp_hbm, sc_v)
    scv = sc_v[...]
    s_val = [scv[k] for k in range(K)]

    def gather_copies(g, slot):
      tbase = wid * TW + g * CHUNK
      copies = []
      for j in range(SUB):
        copies.append(pltpu.make_async_copy(
            w_hbm.at[idw_v.at[slot, pl.ds(j * 128, 128)]],
            rows_v.at[slot, pl.ds(j * 128, 128)], sems[slot]))
        copies.append(pltpu.make_async_copy(
            mix_hbm.at[idq_v.at[slot, pl.ds(j * 128, 128)]],
            mixr_v.at[slot, pl.ds(j * 128, 128)], sems[slot]))
      return tbase, copies

    def fire(g, slot):
      tbase, copies = gather_copies(g, slot)
      pltpu.sync_copy(ids_hbm.at[pl.ds(tbase, CHUNK)], idx_v.at[slot])

      def shift_body(i, carry):
        sl = pl.ds(i * LANES, LANES)
        raw = idx_v[slot, sl]
        idw_v[slot, sl] = ((raw & -1024)
                           + lax.shift_left((raw & 511), 1)
                           + (lax.shift_right_logical(raw, 9) & 1))
        idq_v[slot, sl] = lax.shift_right_logical(raw, 2)
        return carry

      lax.fori_loop(0, GROUPS, shift_body, 0)
      for c in copies:
        c.start()

    def drain(g, slot):
      _, copies = gather_copies(g, slot)
      for c in copies:
        c.wait()

    def compute(g, slot):
      rows = rows_v.at[slot]
      mixr = mixr_v.at[slot]

      def group_body(i, carry):
        t0 = i * LANES
        tok = t0 + lax.iota(jnp.int32, LANES)
        idvec = idx_v[slot, pl.ds(i * LANES, LANES)]
        colb = (idvec & 3) * K
        logits = [plsc.load_gather(mixr, [tok, colb + k]) for k in range(K)]
        m = logits[0]
        for k in range(1, K):
          m = jnp.maximum(m, logits[k])
        e = [jnp.exp(logits[k] - m) for k in range(K)]
        tot = e[0]
        for k in range(1, K):
          tot = tot + e[k]
        inv = 1.0 / tot
        wk = [e[k] * inv * s_val[k] for k in range(K)]
        for i2 in range(LANES):
          t = t0 + i2
          for k in range(K):
            wv = _lane_bcast(wk[k], i2)
            seg = rows[t, pl.ds(k * LANES, LANES)]
            rows[t, pl.ds(k * LANES, LANES)] = seg * wv
        return carry

      lax.fori_loop(0, GROUPS, group_body, 0)
      base_t = wid * TW + g * CHUNK
      pltpu.sync_copy(rows, out_hbm.at[pl.ds(base_t, CHUNK)])

    fire(0, 0)
    for g in range(NCHUNK):
      slot = g % 2
      if g + 1 < NCHUNK:
        fire(g + 1, (g + 1) % 2)
      drain(g, slot)
      compute(g, slot)

  return sc_kernel


def kernel(ids, W, mix, sc):
  B, L = ids.shape
  V, K, d = W.shape
  D = K * d
  N = B * L
  assert V % 4 == 0 and K == 4
  ids_flat = ids.reshape(-1).astype(jnp.int32)
  Wt = jnp.transpose(W, (1, 2, 0)).reshape(D, V)
  Wrp = _tc_repack(Wt, V, D)
  W2 = Wrp.reshape(Wrp.shape[0] * 2, D)
  mix16 = mix.reshape(V // 4, 4 * K)
  scp = jnp.zeros((LANES,), jnp.float32).at[:K].set(
      sc.astype(jnp.float32) * math.sqrt(D))
  out = _make_sc_kernel(N, V, K, D)(ids_flat, W2, mix16, scp)
  return out.reshape(B, L, D)
